# Initial kernel scaffold; baseline (speedup 1.0000x reference)
import jax
import jax.numpy as jnp
from jax.experimental import pallas as pl


def kernel(x, edge_index, w, batch, batch_num, W1, W2, Wo, Wn, gamma, beta):
    raise NotImplementedError("write your pallas kernel here")


# speedup vs baseline: 4.0819x; 4.0819x over previous
import dataclasses
import functools

import jax
import jax.numpy as jnp
from jax import lax
from jax.experimental import pallas as pl
from jax.experimental.pallas import tpu as pltpu
from jax.experimental.pallas import tpu_sc as plsc

N = 10000
E = 320000
D = 128
EPS = 1e-6
NEG = 0.01

NC = 2
NS = 16
NW = NC * NS
EPW = E // NW
C = 80
NCH = EPW // C
STRIPE = 632
LAST_STRIPE = N - STRIPE * (NS - 1)


def _leaky(a):
    return jnp.maximum(a, NEG * a)


def _mlp_body(x_ref, w1t_ref, w2t_ref, h_ref):
    a = jnp.dot(x_ref[...], w1t_ref[...], preferred_element_type=jnp.float32)
    h_ref[...] = jnp.dot(_leaky(a), w2t_ref[...],
                         preferred_element_type=jnp.float32)


def _mlp(x, w1t, w2t):
    return pl.pallas_call(
        _mlp_body,
        out_shape=jax.ShapeDtypeStruct((N, D), jnp.float32),
    )(x, w1t, w2t)


def _sc_agg_body(h_hbm, src_hbm, dst_hbm, w_hbm, z_hbm, out_hbm,
                 acc, srcb, dstb, wb, rows):
    cid = lax.axis_index("c")
    sid = lax.axis_index("s")
    wid = sid * NC + cid

    off = sid * STRIPE

    @pl.when(sid < NS - 1)
    def _():
        s = pl.ds(off, STRIPE)
        pltpu.sync_copy(z_hbm.at[s], acc.at[s])

    @pl.when(sid == NS - 1)
    def _():
        s = pl.ds(off, LAST_STRIPE)
        pltpu.sync_copy(z_hbm.at[s], acc.at[s])

    plsc.subcore_barrier()

    base0 = wid * EPW

    @pl.loop(0, NCH)
    def _chunk(g):
        base = base0 + g * C
        pltpu.sync_copy(src_hbm.at[pl.ds(base, C)], srcb)
        pltpu.sync_copy(dst_hbm.at[pl.ds(base, C)], dstb.at[0])
        pltpu.sync_copy(w_hbm.at[pl.ds(base, C)], wb)
        pltpu.sync_copy(h_hbm.at[srcb], rows)

        @pl.loop(0, C)
        def _edge(e):
            ws = plsc.load_gather(wb, [jnp.full((16,), e, jnp.int32)])
            for k in range(D // 16):
                sl = (e, pl.ds(k * 16, 16))
                rows[sl] = rows[sl] * ws

        pltpu.sync_copy(rows, acc.at[dstb.at[0]], add=True)

    plsc.subcore_barrier()
    plsc.subcore_barrier()

    @pl.when(sid < NS - 1)
    def _():
        s = pl.ds(off, STRIPE)
        pltpu.sync_copy(acc.at[s], out_hbm.at[cid, s])

    @pl.when(sid == NS - 1)
    def _():
        s = pl.ds(off, LAST_STRIPE)
        pltpu.sync_copy(acc.at[s], out_hbm.at[cid, s])


@functools.cache
def _sc_agg_kernel():
    cp = pltpu.CompilerParams()
    if "needs_layout_passes" in pltpu.CompilerParams.__dataclass_fields__:
        cp = dataclasses.replace(cp, needs_layout_passes=False)
    return pl.kernel(
        _sc_agg_body,
        compiler_params=cp,
        mesh=plsc.VectorSubcoreMesh(core_axis_name="c", subcore_axis_name="s"),
        out_type=jax.ShapeDtypeStruct((NC, N, D), jnp.float32),
        scratch_types=[
            pltpu.VMEM_SHARED((N, D), jnp.float32),
            pltpu.VMEM((C,), jnp.int32),
            pltpu.VMEM((1, C), jnp.int32),
            pltpu.VMEM((C,), jnp.float32),
            pltpu.VMEM((C, D), jnp.float32),
        ],
    )


def _finish_body(h_ref, p0_ref, p1_ref, wot_ref, wnt_ref, g_ref, b_ref, o_ref):
    xn = _leaky(p0_ref[...] + p1_ref[...])
    y = (jnp.dot(h_ref[...], wot_ref[...], preferred_element_type=jnp.float32)
         + jnp.dot(xn, wnt_ref[...], preferred_element_type=jnp.float32))
    mu = jnp.sum(y, axis=0, keepdims=True) * (1.0 / N)
    d = y - mu
    var = jnp.sum(d * d, axis=0, keepdims=True) * (1.0 / (N - 1))
    o_ref[...] = d / (jnp.sqrt(var) + EPS) * g_ref[...] + b_ref[...]


def _finish(h, p0, p1, wot, wnt, gamma, beta):
    return pl.pallas_call(
        _finish_body,
        out_shape=jax.ShapeDtypeStruct((N, D), jnp.float32),
    )(h, p0, p1, wot, wnt, gamma, beta)


def kernel(x, edge_index, w, batch, batch_num, W1, W2, Wo, Wn, gamma, beta):
    src = edge_index[1]
    dst = edge_index[0]
    wf = jnp.reshape(w, (E,))
    h = _mlp(x, W1.T, W2.T)
    z = jnp.zeros((N, D), jnp.float32)
    parts = _sc_agg_kernel()(h, src, dst, wf, z)
    return _finish(h, parts[0], parts[1], Wo.T, Wn.T,
                   jnp.reshape(gamma, (1, D)), jnp.reshape(beta, (1, D)))


# speedup vs baseline: 8.2901x; 2.0310x over previous
import dataclasses
import functools

import jax
import jax.numpy as jnp
from jax import lax
from jax.experimental import pallas as pl
from jax.experimental.pallas import tpu as pltpu
from jax.experimental.pallas import tpu_sc as plsc

N = 10000
E = 320000
D = 128
EPS = 1e-6
NEG = 0.01

NC = 2
NS = 16
NW = NC * NS
EPW = E // NW
C = 80
NCH = EPW // C
NITER = (NCH - 3) // 2
STRIPE = 632
LAST_STRIPE = N - STRIPE * (NS - 1)


def _leaky(a):
    return jnp.maximum(a, NEG * a)


def _mlp_body(x_ref, w1t_ref, w2t_ref, h_ref):
    a = jnp.dot(x_ref[...], w1t_ref[...], preferred_element_type=jnp.float32)
    h_ref[...] = jnp.dot(_leaky(a), w2t_ref[...],
                         preferred_element_type=jnp.float32)


def _mlp(x, w1t, w2t):
    return pl.pallas_call(
        _mlp_body,
        out_shape=jax.ShapeDtypeStruct((N, D), jnp.float32),
    )(x, w1t, w2t)


def _sc_agg_body(h_hbm, src_hbm, dst_hbm, w_hbm, z_hbm, out_hbm,
                 acc, isrc, idst, wbuf, rows0, rows1,
                 sem_i, sem_g0, sem_g1, sem_s0, sem_s1, sem_d0, sem_d1):
    cid = lax.axis_index("c")
    sid = lax.axis_index("s")
    wid = sid * NC + cid
    rows = (rows0, rows1)
    sem_g = (sem_g0, sem_g1)
    sem_s = (sem_s0, sem_s1)
    sem_d = (sem_d0, sem_d1)

    off = sid * STRIPE

    @pl.when(sid < NS - 1)
    def _():
        s = pl.ds(off, STRIPE)
        pltpu.sync_copy(z_hbm.at[s], acc.at[s])

    @pl.when(sid == NS - 1)
    def _():
        s = pl.ds(off, LAST_STRIPE)
        pltpu.sync_copy(z_hbm.at[s], acc.at[s])

    blk = pl.ds(wid * EPW, EPW)
    pltpu.async_copy(src_hbm.at[blk], isrc, sem_i).wait()
    pltpu.async_copy(w_hbm.at[blk], wbuf, sem_i).wait()
    plsc.subcore_barrier()

    base0 = wid * EPW

    def start_gather(g, b):
        pltpu.async_copy(dst_hbm.at[pl.ds(base0 + g * C, C)], idst.at[b],
                         sem_d[b])
        pltpu.async_copy(h_hbm.at[isrc.at[pl.ds(g * C, C)]], rows[b], sem_g[b])

    def wait_gather(g, b):
        pltpu.make_async_copy(h_hbm.at[isrc.at[pl.ds(g * C, C)]], rows[b],
                              sem_g[b]).wait()

    def multiply(g, b):
        rb = rows[b]
        base = g * C

        @pl.loop(0, C)
        def _edge(e):
            ws = plsc.load_gather(wbuf, [jnp.full((16,), base + e, jnp.int32)])
            for k in range(D // 16):
                sl = (e, pl.ds(k * 16, 16))
                rb[sl] = rb[sl] * ws

    def start_scatter(g, b):
        pltpu.make_async_copy(dst_hbm.at[pl.ds(base0 + g * C, C)], idst.at[b],
                              sem_d[b]).wait()
        pltpu.async_copy(rows[b], acc.at[idst.at[b]], sem_s[b], add=True)

    def wait_scatter(g, b):
        pltpu.make_async_copy(rows[b], acc.at[idst.at[b]], sem_s[b]).wait()

    start_gather(0, 0)
    start_gather(1, 1)

    @pl.loop(0, NITER)
    def _pair(i):
        g0 = 2 * i
        wait_gather(g0, 0)
        multiply(g0, 0)
        start_scatter(g0, 0)
        wait_gather(g0 + 1, 1)
        multiply(g0 + 1, 1)
        start_scatter(g0 + 1, 1)
        wait_scatter(g0, 0)
        start_gather(g0 + 2, 0)
        wait_scatter(g0 + 1, 1)
        start_gather(g0 + 3, 1)

    g = 2 * NITER
    wait_gather(g, 0)
    multiply(g, 0)
    start_scatter(g, 0)
    wait_gather(g + 1, 1)
    multiply(g + 1, 1)
    start_scatter(g + 1, 1)
    wait_scatter(g, 0)
    start_gather(g + 2, 0)
    wait_gather(g + 2, 0)
    multiply(g + 2, 0)
    start_scatter(g + 2, 0)
    wait_scatter(g + 1, 1)
    wait_scatter(g + 2, 0)

    plsc.subcore_barrier()
    plsc.subcore_barrier()

    @pl.when(sid < NS - 1)
    def _():
        s = pl.ds(off, STRIPE)
        pltpu.sync_copy(acc.at[s], out_hbm.at[cid, s])

    @pl.when(sid == NS - 1)
    def _():
        s = pl.ds(off, LAST_STRIPE)
        pltpu.sync_copy(acc.at[s], out_hbm.at[cid, s])


@functools.cache
def _sc_agg_kernel():
    cp = pltpu.CompilerParams()
    if "needs_layout_passes" in pltpu.CompilerParams.__dataclass_fields__:
        cp = dataclasses.replace(cp, needs_layout_passes=False)
    return pl.kernel(
        _sc_agg_body,
        compiler_params=cp,
        mesh=plsc.VectorSubcoreMesh(core_axis_name="c", subcore_axis_name="s"),
        out_type=jax.ShapeDtypeStruct((NC, N, D), jnp.float32),
        scratch_types=[
            pltpu.VMEM_SHARED((N, D), jnp.float32),
            pltpu.VMEM((EPW,), jnp.int32),
            pltpu.VMEM((2, C), jnp.int32),
            pltpu.VMEM((EPW,), jnp.float32),
            pltpu.VMEM((C, D), jnp.float32),
            pltpu.VMEM((C, D), jnp.float32),
            pltpu.SemaphoreType.DMA,
            pltpu.SemaphoreType.DMA,
            pltpu.SemaphoreType.DMA,
            pltpu.SemaphoreType.DMA,
            pltpu.SemaphoreType.DMA,
            pltpu.SemaphoreType.DMA,
            pltpu.SemaphoreType.DMA,
        ],
    )


def _finish_body(h_ref, p0_ref, p1_ref, wot_ref, wnt_ref, g_ref, b_ref, o_ref):
    xn = _leaky(p0_ref[...] + p1_ref[...])
    y = (jnp.dot(h_ref[...], wot_ref[...], preferred_element_type=jnp.float32)
         + jnp.dot(xn, wnt_ref[...], preferred_element_type=jnp.float32))
    mu = jnp.sum(y, axis=0, keepdims=True) * (1.0 / N)
    d = y - mu
    var = jnp.sum(d * d, axis=0, keepdims=True) * (1.0 / (N - 1))
    o_ref[...] = d / (jnp.sqrt(var) + EPS) * g_ref[...] + b_ref[...]


def _finish(h, p0, p1, wot, wnt, gamma, beta):
    return pl.pallas_call(
        _finish_body,
        out_shape=jax.ShapeDtypeStruct((N, D), jnp.float32),
    )(h, p0, p1, wot, wnt, gamma, beta)


def kernel(x, edge_index, w, batch, batch_num, W1, W2, Wo, Wn, gamma, beta):
    src = edge_index[1]
    dst = edge_index[0]
    wf = jnp.reshape(w, (E,))
    h = _mlp(x, W1.T, W2.T)
    z = jnp.zeros((N, D), jnp.float32)
    parts = _sc_agg_kernel()(h, src, dst, wf, z)
    return _finish(h, parts[0], parts[1], Wo.T, Wn.T,
                   jnp.reshape(gamma, (1, D)), jnp.reshape(beta, (1, D)))


# speedup vs baseline: 11.0781x; 1.3363x over previous
import dataclasses
import functools

import jax
import jax.numpy as jnp
from jax import lax
from jax.experimental import pallas as pl
from jax.experimental.pallas import tpu as pltpu
from jax.experimental.pallas import tpu_sc as plsc

N = 10000
E = 320000
D = 128
EPS = 1e-6
NEG = 0.01

NC = 2
NS = 16
NW = NC * NS
EPW = E // NW
C = 80
NCH = EPW // C
DEPTH = 4
DDEPTH = 2 * DEPTH
SKEW = 2
NMAIN = 120
STRIPE = 632
LAST_STRIPE = N - STRIPE * (NS - 1)


def _leaky(a):
    return jnp.maximum(a, NEG * a)


def _mlp_body(x_ref, w1t_ref, w2t_ref, h_ref):
    a = jnp.dot(x_ref[...], w1t_ref[...], preferred_element_type=jnp.float32)
    h_ref[...] = jnp.dot(_leaky(a), w2t_ref[...],
                         preferred_element_type=jnp.float32)


def _mlp(x, w1t, w2t):
    return pl.pallas_call(
        _mlp_body,
        out_shape=jax.ShapeDtypeStruct((N, D), jnp.float32),
    )(x, w1t, w2t)


def _sc_agg_body(h_hbm, src_hbm, dst_hbm, w_hbm, z_hbm, out_hbm,
                 acc, ibs, ibd, ibw, rows0, rows1, rows2, rows3,
                 sem_is, sem_id, sem_iw, sem_g, sem_s):
    cid = lax.axis_index("c")
    sid = lax.axis_index("s")
    wid = sid * NC + cid
    rows = (rows0, rows1, rows2, rows3)

    off = sid * STRIPE

    @pl.when(sid < NS - 1)
    def _():
        s = pl.ds(off, STRIPE)
        pltpu.sync_copy(z_hbm.at[s], acc.at[s])

    @pl.when(sid == NS - 1)
    def _():
        s = pl.ds(off, LAST_STRIPE)
        pltpu.sync_copy(z_hbm.at[s], acc.at[s])

    plsc.subcore_barrier()

    base0 = wid * EPW

    def csl(g):
        return pl.ds(base0 + g * C, C)

    def start_idx(g, b):
        pltpu.async_copy(src_hbm.at[csl(g)], ibs.at[b], sem_is.at[b])
        pltpu.async_copy(dst_hbm.at[csl(g)], ibd.at[g & 7], sem_id.at[b])
        pltpu.async_copy(w_hbm.at[csl(g)], ibw.at[b], sem_iw.at[b])

    def a_stage(g, b, first=False):
        if not first:
            pltpu.make_async_copy(rows[b], acc.at[ibd.at[(g - DEPTH) & 7]],
                                  sem_s.at[b]).wait()
        pltpu.make_async_copy(src_hbm.at[csl(g)], ibs.at[b],
                              sem_is.at[b]).wait()
        pltpu.async_copy(h_hbm.at[ibs.at[b]], rows[b], sem_g.at[b])

    def b_stage(g, b, prefetch=True):
        rb = rows[b]
        pltpu.make_async_copy(h_hbm.at[ibs.at[b]], rb, sem_g.at[b]).wait()
        pltpu.make_async_copy(w_hbm.at[csl(g)], ibw.at[b], sem_iw.at[b]).wait()
        bsel = jnp.full((16,), b, jnp.int32)

        @pl.loop(0, C)
        def _edge(e):
            ws = plsc.load_gather(ibw, [bsel, jnp.full((16,), e, jnp.int32)])
            for k in range(D // 16):
                sl = (e, pl.ds(k * 16, 16))
                rb[sl] = rb[sl] * ws

        pltpu.make_async_copy(dst_hbm.at[csl(g)], ibd.at[g & 7],
                              sem_id.at[b]).wait()
        pltpu.async_copy(rb, acc.at[ibd.at[g & 7]], sem_s.at[b], add=True)
        if prefetch:
            start_idx(g + DEPTH, b)

    for g in range(DEPTH):
        start_idx(g, g)
    a_stage(0, 0, first=True)
    a_stage(1, 1, first=True)
    a_stage(2, 2, first=True)
    b_stage(0, 0)
    a_stage(3, 3, first=True)
    b_stage(1, 1)
    a_stage(4, 0)
    b_stage(2, 2)
    a_stage(5, 1)
    b_stage(3, 3)

    @pl.loop(1, NMAIN // DEPTH)
    def _block(j):
        t0 = DEPTH * j
        for b in range(DEPTH):
            a_stage(t0 + b + SKEW, (b + SKEW) % DEPTH)
            b_stage(t0 + b, b)

    a_stage(122, 2)
    b_stage(120, 0)
    a_stage(123, 3)
    b_stage(121, 1, prefetch=False)
    a_stage(124, 0)
    b_stage(122, 2, prefetch=False)
    b_stage(123, 3, prefetch=False)
    b_stage(124, 0, prefetch=False)
    for g in range(121, 125):
        b = g % DEPTH
        pltpu.make_async_copy(rows[b], acc.at[ibd.at[g & 7]],
                              sem_s.at[b]).wait()

    plsc.subcore_barrier()
    plsc.subcore_barrier()

    @pl.when(sid < NS - 1)
    def _():
        s = pl.ds(off, STRIPE)
        pltpu.sync_copy(acc.at[s], out_hbm.at[cid, s])

    @pl.when(sid == NS - 1)
    def _():
        s = pl.ds(off, LAST_STRIPE)
        pltpu.sync_copy(acc.at[s], out_hbm.at[cid, s])


@functools.cache
def _sc_agg_kernel():
    cp = pltpu.CompilerParams()
    if "needs_layout_passes" in pltpu.CompilerParams.__dataclass_fields__:
        cp = dataclasses.replace(cp, needs_layout_passes=False)
    return pl.kernel(
        _sc_agg_body,
        compiler_params=cp,
        mesh=plsc.VectorSubcoreMesh(core_axis_name="c", subcore_axis_name="s"),
        out_type=jax.ShapeDtypeStruct((NC, N, D), jnp.float32),
        scratch_types=[
            pltpu.VMEM_SHARED((N, D), jnp.float32),
            pltpu.VMEM((DEPTH, C), jnp.int32),
            pltpu.VMEM((DDEPTH, C), jnp.int32),
            pltpu.VMEM((DEPTH, C), jnp.float32),
            pltpu.VMEM((C, D), jnp.float32),
            pltpu.VMEM((C, D), jnp.float32),
            pltpu.VMEM((C, D), jnp.float32),
            pltpu.VMEM((C, D), jnp.float32),
            pltpu.SemaphoreType.DMA((DEPTH,)),
            pltpu.SemaphoreType.DMA((DEPTH,)),
            pltpu.SemaphoreType.DMA((DEPTH,)),
            pltpu.SemaphoreType.DMA((DEPTH,)),
            pltpu.SemaphoreType.DMA((DEPTH,)),
        ],
    )


def _finish_body(h_ref, p0_ref, p1_ref, wot_ref, wnt_ref, g_ref, b_ref, o_ref):
    xn = _leaky(p0_ref[...] + p1_ref[...])
    y = (jnp.dot(h_ref[...], wot_ref[...], preferred_element_type=jnp.float32)
         + jnp.dot(xn, wnt_ref[...], preferred_element_type=jnp.float32))
    mu = jnp.sum(y, axis=0, keepdims=True) * (1.0 / N)
    d = y - mu
    var = jnp.sum(d * d, axis=0, keepdims=True) * (1.0 / (N - 1))
    o_ref[...] = d / (jnp.sqrt(var) + EPS) * g_ref[...] + b_ref[...]


def _finish(h, p0, p1, wot, wnt, gamma, beta):
    return pl.pallas_call(
        _finish_body,
        out_shape=jax.ShapeDtypeStruct((N, D), jnp.float32),
    )(h, p0, p1, wot, wnt, gamma, beta)


def kernel(x, edge_index, w, batch, batch_num, W1, W2, Wo, Wn, gamma, beta):
    src = edge_index[1]
    dst = edge_index[0]
    wf = jnp.reshape(w, (E,))
    h = _mlp(x, W1.T, W2.T)
    z = jnp.zeros((N, D), jnp.float32)
    parts = _sc_agg_kernel()(h, src, dst, wf, z)
    return _finish(h, parts[0], parts[1], Wo.T, Wn.T,
                   jnp.reshape(gamma, (1, D)), jnp.reshape(beta, (1, D)))


# speedup vs baseline: 11.6254x; 1.0494x over previous
import dataclasses
import functools

import jax
import jax.numpy as jnp
from jax import lax
from jax.experimental import pallas as pl
from jax.experimental.pallas import tpu as pltpu
from jax.experimental.pallas import tpu_sc as plsc

N = 10000
E = 320000
D = 128
EPS = 1e-6
NEG = 0.01

NC = 2
NS = 16
NW = NC * NS
EPW = E // NW
C = 80
NCH = EPW // C
DEPTH = 4
DDEPTH = 2 * DEPTH
SKEW = 2
NMAIN = 120
STRIPE = 632
LAST_STRIPE = N - STRIPE * (NS - 1)


def _leaky(a):
    return jnp.maximum(a, NEG * a)


def _mlp_body(x_ref, w1t_ref, w2t_ref, wot_ref, h_ref, t_ref):
    a = jnp.dot(x_ref[...], w1t_ref[...], preferred_element_type=jnp.float32)
    h = jnp.dot(_leaky(a), w2t_ref[...], preferred_element_type=jnp.float32)
    h_ref[...] = h
    t_ref[...] = jnp.dot(h, wot_ref[...], preferred_element_type=jnp.float32)


def _mlp(x, w1t, w2t, wot):
    return pl.pallas_call(
        _mlp_body,
        out_shape=(jax.ShapeDtypeStruct((N, D), jnp.float32),
                   jax.ShapeDtypeStruct((N, D), jnp.float32)),
    )(x, w1t, w2t, wot)


def _sc_agg_body(h_hbm, src_hbm, dst_hbm, w_hbm, z_hbm, out0_hbm, out1_hbm,
                 acc, ibs, ibd, ibw, rows0, rows1, rows2, rows3,
                 sem_is, sem_id, sem_iw, sem_g, sem_s):
    cid = lax.axis_index("c")
    sid = lax.axis_index("s")
    wid = sid * NC + cid
    rows = (rows0, rows1, rows2, rows3)

    off = sid * STRIPE

    @pl.when(sid < NS - 1)
    def _():
        s = pl.ds(off, STRIPE)
        pltpu.sync_copy(z_hbm.at[s], acc.at[s])

    @pl.when(sid == NS - 1)
    def _():
        s = pl.ds(off, LAST_STRIPE)
        pltpu.sync_copy(z_hbm.at[s], acc.at[s])

    plsc.subcore_barrier()

    base0 = wid * EPW

    def csl(g):
        return pl.ds(base0 + g * C, C)

    def start_idx(g, b):
        pltpu.async_copy(src_hbm.at[csl(g)], ibs.at[b], sem_is.at[b])
        pltpu.async_copy(dst_hbm.at[csl(g)], ibd.at[g & 7], sem_id.at[b])
        pltpu.async_copy(w_hbm.at[csl(g)], ibw.at[b], sem_iw.at[b])

    def a_stage(g, b, first=False):
        if not first:
            pltpu.make_async_copy(rows[b], acc.at[ibd.at[(g - DEPTH) & 7]],
                                  sem_s.at[b]).wait()
        pltpu.make_async_copy(src_hbm.at[csl(g)], ibs.at[b],
                              sem_is.at[b]).wait()
        pltpu.async_copy(h_hbm.at[ibs.at[b]], rows[b], sem_g.at[b])

    def b_stage(g, b, prefetch=True):
        rb = rows[b]
        pltpu.make_async_copy(h_hbm.at[ibs.at[b]], rb, sem_g.at[b]).wait()
        pltpu.make_async_copy(w_hbm.at[csl(g)], ibw.at[b], sem_iw.at[b]).wait()
        bsel = jnp.full((16,), b, jnp.int32)

        @plsc.parallel_loop(0, C, unroll=2)
        def _edge(e):
            ws = plsc.load_gather(ibw, [bsel, jnp.full((16,), e, jnp.int32)])
            for k in range(D // 16):
                sl = (e, pl.ds(k * 16, 16))
                rb[sl] = rb[sl] * ws

        pltpu.make_async_copy(dst_hbm.at[csl(g)], ibd.at[g & 7],
                              sem_id.at[b]).wait()
        pltpu.async_copy(rb, acc.at[ibd.at[g & 7]], sem_s.at[b], add=True)
        if prefetch:
            start_idx(g + DEPTH, b)

    for g in range(DEPTH):
        start_idx(g, g)
    a_stage(0, 0, first=True)
    a_stage(1, 1, first=True)
    a_stage(2, 2, first=True)
    b_stage(0, 0)
    a_stage(3, 3, first=True)
    b_stage(1, 1)
    a_stage(4, 0)
    b_stage(2, 2)
    a_stage(5, 1)
    b_stage(3, 3)

    @pl.loop(1, NMAIN // DEPTH)
    def _block(j):
        t0 = DEPTH * j
        for b in range(DEPTH):
            a_stage(t0 + b + SKEW, (b + SKEW) % DEPTH)
            b_stage(t0 + b, b)

    a_stage(122, 2)
    b_stage(120, 0)
    a_stage(123, 3)
    b_stage(121, 1, prefetch=False)
    a_stage(124, 0)
    b_stage(122, 2, prefetch=False)
    b_stage(123, 3, prefetch=False)
    b_stage(124, 0, prefetch=False)
    for g in range(121, 125):
        b = g % DEPTH
        pltpu.make_async_copy(rows[b], acc.at[ibd.at[g & 7]],
                              sem_s.at[b]).wait()

    plsc.subcore_barrier()
    plsc.subcore_barrier()

    sz = sid * STRIPE
    for c, out_hbm in ((0, out0_hbm), (1, out1_hbm)):
        @pl.when((cid == c) & (sid < NS - 1))
        def _():
            s = pl.ds(sz, STRIPE)
            pltpu.sync_copy(acc.at[s], out_hbm.at[s])

        @pl.when((cid == c) & (sid == NS - 1))
        def _():
            s = pl.ds(sz, LAST_STRIPE)
            pltpu.sync_copy(acc.at[s], out_hbm.at[s])


@functools.cache
def _sc_agg_kernel():
    cp = pltpu.CompilerParams()
    if "needs_layout_passes" in pltpu.CompilerParams.__dataclass_fields__:
        cp = dataclasses.replace(cp, needs_layout_passes=False)
    return pl.kernel(
        _sc_agg_body,
        compiler_params=cp,
        mesh=plsc.VectorSubcoreMesh(core_axis_name="c", subcore_axis_name="s"),
        out_type=(jax.ShapeDtypeStruct((N, D), jnp.float32),
                  jax.ShapeDtypeStruct((N, D), jnp.float32)),
        scratch_types=[
            pltpu.VMEM_SHARED((N, D), jnp.float32),
            pltpu.VMEM((DEPTH, C), jnp.int32),
            pltpu.VMEM((DDEPTH, C), jnp.int32),
            pltpu.VMEM((DEPTH, C), jnp.float32),
            pltpu.VMEM((C, D), jnp.float32),
            pltpu.VMEM((C, D), jnp.float32),
            pltpu.VMEM((C, D), jnp.float32),
            pltpu.VMEM((C, D), jnp.float32),
            pltpu.SemaphoreType.DMA((DEPTH,)),
            pltpu.SemaphoreType.DMA((DEPTH,)),
            pltpu.SemaphoreType.DMA((DEPTH,)),
            pltpu.SemaphoreType.DMA((DEPTH,)),
            pltpu.SemaphoreType.DMA((DEPTH,)),
        ],
    )


def _finish_body(t_ref, p0_ref, p1_ref, wnt_ref, g_ref, b_ref, o_ref):
    xn = _leaky(p0_ref[...] + p1_ref[...])
    y = t_ref[...] + jnp.dot(xn, wnt_ref[...],
                             preferred_element_type=jnp.float32)
    mu = jnp.sum(y, axis=0, keepdims=True) * (1.0 / N)
    d = y - mu
    var = jnp.sum(d * d, axis=0, keepdims=True) * (1.0 / (N - 1))
    o_ref[...] = d / (jnp.sqrt(var) + EPS) * g_ref[...] + b_ref[...]


def _finish(t, p0, p1, wnt, gamma, beta):
    return pl.pallas_call(
        _finish_body,
        out_shape=jax.ShapeDtypeStruct((N, D), jnp.float32),
    )(t, p0, p1, wnt, gamma, beta)


def kernel(x, edge_index, w, batch, batch_num, W1, W2, Wo, Wn, gamma, beta):
    src = edge_index[1]
    dst = edge_index[0]
    wf = jnp.reshape(w, (E,))
    h, t = _mlp(x, W1.T, W2.T, Wo.T)
    z = jnp.zeros((N, D), jnp.float32)
    p0, p1 = _sc_agg_kernel()(h, src, dst, wf, z)
    return _finish(t, p0, p1, Wn.T,
                   jnp.reshape(gamma, (1, D)), jnp.reshape(beta, (1, D)))


# speedup vs baseline: 12.3543x; 1.0627x over previous
import dataclasses
import functools

import jax
import jax.numpy as jnp
from jax import lax
from jax.experimental import pallas as pl
from jax.experimental.pallas import tpu as pltpu
from jax.experimental.pallas import tpu_sc as plsc

N = 10000
E = 320000
D = 128
EPS = 1e-6
NEG = 0.01

NC = 2
NS = 16
NW = NC * NS
EPW = E // NW
C = 80
NCH = EPW // C
DEPTH = 4
DDEPTH = 2 * DEPTH
SKEW = 2
NMAIN = 120
STRIPE = 632
LAST_STRIPE = N - STRIPE * (NS - 1)


def _leaky(a):
    return jnp.maximum(a, NEG * a)


def _mlp_body(x_ref, w1t_ref, w2t_ref, wot_ref, h_ref, t_ref):
    a = jnp.dot(x_ref[...], w1t_ref[...], preferred_element_type=jnp.float32)
    h = jnp.dot(_leaky(a), w2t_ref[...], preferred_element_type=jnp.float32)
    h_ref[...] = h
    t_ref[...] = jnp.dot(h, wot_ref[...], preferred_element_type=jnp.float32)


def _mlp(x, w1t, w2t, wot):
    return pl.pallas_call(
        _mlp_body,
        out_shape=(jax.ShapeDtypeStruct((N, D), jnp.float32),
                   jax.ShapeDtypeStruct((N, D), jnp.float32)),
    )(x, w1t, w2t, wot)


def _sc_agg_body(h_hbm, ei_hbm, w_hbm, z_hbm, out0_hbm, out1_hbm,
                 acc, ibs, ibd, ibw, rows0, rows1, rows2, rows3,
                 sem_is, sem_id, sem_iw, sem_g, sem_s):
    cid = lax.axis_index("c")
    sid = lax.axis_index("s")
    wid = sid * NC + cid
    rows = (rows0, rows1, rows2, rows3)

    off = sid * STRIPE

    @pl.when(sid < NS - 1)
    def _():
        s = pl.ds(off, STRIPE)
        pltpu.sync_copy(z_hbm.at[s], acc.at[s])

    @pl.when(sid == NS - 1)
    def _():
        s = pl.ds(off, LAST_STRIPE)
        pltpu.sync_copy(z_hbm.at[s], acc.at[s])

    plsc.subcore_barrier()

    base0 = wid * EPW

    def csl(g):
        return pl.ds(base0 + g * C, C)

    def ssl(g):
        return pl.ds(E + base0 + g * C, C)

    def start_idx(g, b):
        pltpu.async_copy(ei_hbm.at[ssl(g)], ibs.at[b], sem_is.at[b])
        pltpu.async_copy(ei_hbm.at[csl(g)], ibd.at[g & 7], sem_id.at[b])
        pltpu.async_copy(w_hbm.at[csl(g)], ibw.at[b], sem_iw.at[b])

    def a_stage(g, b, first=False):
        if not first:
            pltpu.make_async_copy(rows[b], acc.at[ibd.at[(g - DEPTH) & 7]],
                                  sem_s.at[b]).wait()
        pltpu.make_async_copy(ei_hbm.at[ssl(g)], ibs.at[b],
                              sem_is.at[b]).wait()
        pltpu.async_copy(h_hbm.at[ibs.at[b]], rows[b], sem_g.at[b])

    def b_stage(g, b, prefetch=True):
        rb = rows[b]
        pltpu.make_async_copy(h_hbm.at[ibs.at[b]], rb, sem_g.at[b]).wait()
        pltpu.make_async_copy(w_hbm.at[csl(g)], ibw.at[b],
                              sem_iw.at[b]).wait()
        bsel = jnp.full((16,), b, jnp.int32)

        @plsc.parallel_loop(0, C, unroll=2)
        def _edge(e):
            ws = plsc.load_gather(ibw, [bsel, jnp.full((16,), e, jnp.int32)])
            for k in range(D // 16):
                sl = (e, pl.ds(k * 16, 16))
                rb[sl] = rb[sl] * ws

        pltpu.make_async_copy(ei_hbm.at[csl(g)], ibd.at[g & 7],
                              sem_id.at[b]).wait()
        pltpu.async_copy(rb, acc.at[ibd.at[g & 7]], sem_s.at[b], add=True)
        if prefetch:
            start_idx(g + DEPTH, b)

    for g in range(DEPTH):
        start_idx(g, g)
    a_stage(0, 0, first=True)
    a_stage(1, 1, first=True)
    a_stage(2, 2, first=True)
    b_stage(0, 0)
    a_stage(3, 3, first=True)
    b_stage(1, 1)
    a_stage(4, 0)
    b_stage(2, 2)
    a_stage(5, 1)
    b_stage(3, 3)

    @pl.loop(1, NMAIN // DEPTH)
    def _block(j):
        t0 = DEPTH * j
        for b in range(DEPTH):
            a_stage(t0 + b + SKEW, (b + SKEW) % DEPTH)
            b_stage(t0 + b, b)

    a_stage(122, 2)
    b_stage(120, 0)
    a_stage(123, 3)
    b_stage(121, 1, prefetch=False)
    a_stage(124, 0)
    b_stage(122, 2, prefetch=False)
    b_stage(123, 3, prefetch=False)
    b_stage(124, 0, prefetch=False)
    for g in range(121, 125):
        b = g % DEPTH
        pltpu.make_async_copy(rows[b], acc.at[ibd.at[g & 7]],
                              sem_s.at[b]).wait()

    plsc.subcore_barrier()
    plsc.subcore_barrier()

    sz = sid * STRIPE
    for c, out_hbm in ((0, out0_hbm), (1, out1_hbm)):
        @pl.when((cid == c) & (sid < NS - 1))
        def _():
            s = pl.ds(sz, STRIPE)
            pltpu.sync_copy(acc.at[s], out_hbm.at[s])

        @pl.when((cid == c) & (sid == NS - 1))
        def _():
            s = pl.ds(sz, LAST_STRIPE)
            pltpu.sync_copy(acc.at[s], out_hbm.at[s])


@functools.cache
def _sc_agg_kernel():
    cp = pltpu.CompilerParams()
    if "needs_layout_passes" in pltpu.CompilerParams.__dataclass_fields__:
        cp = dataclasses.replace(cp, needs_layout_passes=False)
    return pl.kernel(
        _sc_agg_body,
        compiler_params=cp,
        mesh=plsc.VectorSubcoreMesh(core_axis_name="c", subcore_axis_name="s"),
        out_type=(jax.ShapeDtypeStruct((N, D), jnp.float32),
                  jax.ShapeDtypeStruct((N, D), jnp.float32)),
        scratch_types=[
            pltpu.VMEM_SHARED((N, D), jnp.float32),
            pltpu.VMEM((DEPTH, C), jnp.int32),
            pltpu.VMEM((DDEPTH, C), jnp.int32),
            pltpu.VMEM((DEPTH, C), jnp.float32),
            pltpu.VMEM((C, D), jnp.float32),
            pltpu.VMEM((C, D), jnp.float32),
            pltpu.VMEM((C, D), jnp.float32),
            pltpu.VMEM((C, D), jnp.float32),
            pltpu.SemaphoreType.DMA((DEPTH,)),
            pltpu.SemaphoreType.DMA((DEPTH,)),
            pltpu.SemaphoreType.DMA((DEPTH,)),
            pltpu.SemaphoreType.DMA((DEPTH,)),
            pltpu.SemaphoreType.DMA((DEPTH,)),
        ],
    )


def _finish_body(t_ref, p0_ref, p1_ref, wnt_ref, g_ref, b_ref, o_ref):
    xn = _leaky(p0_ref[...] + p1_ref[...])
    y = t_ref[...] + jnp.dot(xn, wnt_ref[...],
                             preferred_element_type=jnp.float32)
    mu = jnp.sum(y, axis=0, keepdims=True) * (1.0 / N)
    d = y - mu
    var = jnp.sum(d * d, axis=0, keepdims=True) * (1.0 / (N - 1))
    gam = jnp.reshape(g_ref[...], (1, D))
    bet = jnp.reshape(b_ref[...], (1, D))
    o_ref[...] = d / (jnp.sqrt(var) + EPS) * gam + bet


def _finish(t, p0, p1, wnt, gamma, beta):
    return pl.pallas_call(
        _finish_body,
        out_shape=jax.ShapeDtypeStruct((N, D), jnp.float32),
    )(t, p0, p1, wnt, gamma, beta)


def kernel(x, edge_index, w, batch, batch_num, W1, W2, Wo, Wn, gamma, beta):
    eflat = jnp.reshape(edge_index, (2 * E,))
    wf = jnp.reshape(w, (E,))
    h, t = _mlp(x, W1.T, W2.T, Wo.T)
    z = jnp.zeros((N, D), jnp.float32)
    p0, p1 = _sc_agg_kernel()(h, eflat, wf, z)
    return _finish(t, p0, p1, Wn.T, gamma, beta)
